# Initial kernel scaffold; baseline (speedup 1.0000x reference)
import jax
import jax.numpy as jnp
from jax.experimental import pallas as pl


def kernel(x, W_exp, b_exp, W_gate, b_gate):
    raise NotImplementedError("write your pallas kernel here")


# speedup vs baseline: 1.1145x; 1.1145x over previous
import jax
import jax.numpy as jnp
from jax import lax
from jax.experimental import pallas as pl
from jax.experimental.pallas import tpu as pltpu

D_MODEL = 2048
N_EXP = 16
EPAD = 128
BT = 512


def _fused_body(x_ref, we_ref, be_ref, wg_ref, bg_ref, out_ref, idx_ref, w_ref):
    xb = x_ref[...]
    gl = lax.dot_general(xb, wg_ref[...], (((1,), (1,)), ((), ())),
                         preferred_element_type=jnp.float32)
    gl = gl + bg_ref[...]
    iota = lax.broadcasted_iota(jnp.int32, (BT, EPAD), 1)
    m0 = jnp.max(gl, axis=1, keepdims=True)
    i0 = jnp.min(jnp.where(gl == m0, iota, EPAD), axis=1, keepdims=True)
    gl2 = jnp.where(iota == i0, -jnp.inf, gl)
    m1 = jnp.max(gl2, axis=1, keepdims=True)
    i1 = jnp.min(jnp.where(gl2 == m1, iota, EPAD), axis=1, keepdims=True)
    e1 = jnp.exp(m1 - m0)
    s = 1.0 + e1
    w0 = 1.0 / s
    w1 = e1 / s
    idx_ref[...] = jnp.where(iota == 0, i0, jnp.where(iota == 1, i1, 0))
    w_ref[...] = jnp.where(iota == 0, w0, jnp.where(iota == 1, w1, 0.0))
    acc = lax.dot_general(xb, we_ref[...], (((1,), (1,)), ((), ())),
                          preferred_element_type=jnp.float32)
    out_ref[...] = (acc + be_ref[...]) * (w0 + w1)


def kernel(x, W_exp, b_exp, W_gate, b_gate):
    n_tok = x.shape[0]
    wg_pad = jnp.pad(W_gate, ((0, EPAD - N_EXP), (0, 0)))
    bg_pad = jnp.pad(b_gate, (0, EPAD - N_EXP), constant_values=-1e30)
    bg_pad = bg_pad.reshape(1, EPAD)
    be2 = b_exp.reshape(1, D_MODEL)

    grid = (n_tok // BT,)
    out, idxp, wp = pl.pallas_call(
        _fused_body,
        grid=grid,
        in_specs=[
            pl.BlockSpec((BT, D_MODEL), lambda i: (i, 0)),
            pl.BlockSpec((D_MODEL, D_MODEL), lambda i: (0, 0)),
            pl.BlockSpec((1, D_MODEL), lambda i: (0, 0)),
            pl.BlockSpec((EPAD, D_MODEL), lambda i: (0, 0)),
            pl.BlockSpec((1, EPAD), lambda i: (0, 0)),
        ],
        out_specs=[
            pl.BlockSpec((BT, D_MODEL), lambda i: (i, 0)),
            pl.BlockSpec((BT, EPAD), lambda i: (i, 0)),
            pl.BlockSpec((BT, EPAD), lambda i: (i, 0)),
        ],
        out_shape=[
            jax.ShapeDtypeStruct((n_tok, D_MODEL), jnp.float32),
            jax.ShapeDtypeStruct((n_tok, EPAD), jnp.int32),
            jax.ShapeDtypeStruct((n_tok, EPAD), jnp.float32),
        ],
    )(x, W_exp, be2, wg_pad, bg_pad)
    return out, idxp[:, :2], wp[:, :2]
